# baseline (speedup 1.0000x reference)
import functools

import jax
import jax.numpy as jnp
from jax import lax
from jax.experimental import pallas as pl
from jax.experimental.pallas import tpu as pltpu
from jax.experimental.pallas import tpu_sc as plsc

D_MODEL = 1024
D_FF = 4096
E = 8
TOP_K = 2
T = 2048
CAPACITY = 640
NSLOT = E * CAPACITY
TK = T * TOP_K

NW = 32
SPT = NSLOT // NW
TPT = T // NW

RBLK = 128
NRB = T // RBLK

FBLK = 2048
NFB = D_FF // FBLK


def _router_body(x_ref, wg_ref, slotd_ref, slotc_ref, gate_ref, carry_ref):
    i = pl.program_id(0)

    @pl.when(i == 0)
    def _():
        carry_ref[...] = jnp.zeros_like(carry_ref)

    xb = x_ref[...]
    wg = wg_ref[...]
    logits = jnp.dot(xb, wg, preferred_element_type=jnp.float32)
    probs = jax.nn.softmax(logits, axis=-1)

    ids = lax.broadcasted_iota(jnp.int32, (RBLK, E), 1)
    m0 = jnp.max(probs, axis=-1, keepdims=True)
    am0 = jnp.min(jnp.where(probs == m0, ids, E), axis=-1, keepdims=True)
    probs2 = jnp.where(ids == am0, -1.0, probs)
    m1 = jnp.max(probs2, axis=-1, keepdims=True)
    am1 = jnp.min(jnp.where(probs2 == m1, ids, E), axis=-1, keepdims=True)
    gsum = m0 + m1 + 1e-9
    g0 = m0 / gsum
    g1 = m1 / gsum

    oh0 = (ids == am0).astype(jnp.float32)
    oh1 = (ids == am1).astype(jnp.float32)
    cnt = oh0 + oh1

    r = lax.broadcasted_iota(jnp.int32, (RBLK, RBLK), 0)
    c = lax.broadcasted_iota(jnp.int32, (RBLK, RBLK), 1)
    tri = (r > c).astype(jnp.float32)
    base = jnp.dot(tri, cnt, preferred_element_type=jnp.float32)
    base = base + carry_ref[...]
    carry_ref[...] = carry_ref[...] + jnp.sum(cnt, axis=0, keepdims=True)

    pos0 = jnp.sum(base * oh0, axis=-1, keepdims=True).astype(jnp.int32)
    pos1 = jnp.sum(base * oh1, axis=-1, keepdims=True).astype(jnp.int32)
    keep0 = pos0 < CAPACITY
    keep1 = pos1 < CAPACITY
    slot0 = am0 * CAPACITY + pos0
    slot1 = am1 * CAPACITY + pos1

    slotd_ref[...] = jnp.concatenate(
        [jnp.where(keep0, slot0, -1), jnp.where(keep1, slot1, -1)], axis=1)
    slotc_ref[...] = jnp.concatenate(
        [jnp.where(keep0, slot0, 0), jnp.where(keep1, slot1, 0)], axis=1)
    gate_ref[...] = jnp.concatenate(
        [g0 * keep0.astype(jnp.float32), g1 * keep1.astype(jnp.float32)], axis=1)


def _router(x, Wg):
    return pl.pallas_call(
        _router_body,
        grid=(NRB,),
        in_specs=[
            pl.BlockSpec((RBLK, D_MODEL), lambda i: (i, 0)),
            pl.BlockSpec((D_MODEL, E), lambda i: (0, 0)),
        ],
        out_specs=[
            pl.BlockSpec((RBLK, TOP_K), lambda i: (i, 0)),
            pl.BlockSpec((RBLK, TOP_K), lambda i: (i, 0)),
            pl.BlockSpec((RBLK, TOP_K), lambda i: (i, 0)),
        ],
        out_shape=[
            jax.ShapeDtypeStruct((T, TOP_K), jnp.int32),
            jax.ShapeDtypeStruct((T, TOP_K), jnp.int32),
            jax.ShapeDtypeStruct((T, TOP_K), jnp.float32),
        ],
        scratch_shapes=[pltpu.VMEM((1, E), jnp.float32)],
    )(x, Wg)


def _dispatch_body(slotd_hbm, x_hbm, out_hbm, slot_v, src_v, rows_v, sem):
    cid = lax.axis_index("c")
    sid = lax.axis_index("s")
    wid = sid * 2 + cid
    lo = wid * SPT

    pltpu.sync_copy(slotd_hbm, slot_v)

    def init_i(i, _):
        src_v[pl.ds(i * 16, 16)] = jnp.zeros((16,), jnp.int32)
        return 0
    lax.fori_loop(0, SPT // 16, init_i, 0)

    def scat_i(i, _):
        sv = slot_v[pl.ds(i * 16, 16)]
        idx = sv - lo
        m = (idx >= 0) & (idx < SPT)
        idxs = jnp.where(m, idx, 0)
        tvec = lax.shift_right_logical(i * 16 + lax.iota(jnp.int32, 16), 1)
        plsc.store_scatter(src_v, [idxs], tvec, mask=m)
        return 0
    lax.fori_loop(0, TK // 16, scat_i, 0)

    for j in range(2):
        pltpu.async_copy(
            x_hbm.at[src_v.at[pl.ds(j * 80, 80)]], rows_v, sem).wait()
        pltpu.sync_copy(rows_v, out_hbm.at[pl.ds(lo + j * 80, 80)])


def _dispatch(slotd_flat, x):
    mesh = plsc.VectorSubcoreMesh(core_axis_name="c", subcore_axis_name="s", num_cores=2, num_subcores=16)
    return pl.kernel(
        _dispatch_body,
        out_type=jax.ShapeDtypeStruct((NSLOT, D_MODEL), jnp.float32),
        mesh=mesh,
        compiler_params=pltpu.CompilerParams(needs_layout_passes=False),
        scratch_types=[
            pltpu.VMEM((TK,), jnp.int32),
            pltpu.VMEM((SPT,), jnp.int32),
            pltpu.VMEM((80, D_MODEL), jnp.float32),
            pltpu.SemaphoreType.DMA,
        ],
    )(slotd_flat, x)


def _ffn_body(a_ref, w1_ref, b1_ref, w2_ref, b2_ref, y_ref, acc_ref):
    f = pl.program_id(1)
    a = a_ref[0]
    h = jnp.dot(a, w1_ref[0], preferred_element_type=jnp.float32)
    h = jnp.maximum(h + b1_ref[0], 0.0)
    hb = h.astype(jnp.bfloat16)
    part = jnp.dot(hb, w2_ref[0], preferred_element_type=jnp.float32)

    @pl.when(f == 0)
    def _():
        acc_ref[...] = part

    @pl.when(f != 0)
    def _():
        acc_ref[...] = acc_ref[...] + part

    @pl.when(f == NFB - 1)
    def _():
        y_ref[0] = acc_ref[...] + b2_ref[0]


def _ffn(bufs_bf, w1b, b1, w2b, b2):
    return pl.pallas_call(
        _ffn_body,
        grid=(E, NFB),
        in_specs=[
            pl.BlockSpec((1, CAPACITY, D_MODEL), lambda e, f: (e, 0, 0)),
            pl.BlockSpec((1, D_MODEL, FBLK), lambda e, f: (e, 0, f)),
            pl.BlockSpec((1, 1, FBLK), lambda e, f: (e, 0, f)),
            pl.BlockSpec((1, FBLK, D_MODEL), lambda e, f: (e, f, 0)),
            pl.BlockSpec((1, 1, D_MODEL), lambda e, f: (e, 0, 0)),
        ],
        out_specs=pl.BlockSpec((1, CAPACITY, D_MODEL), lambda e, f: (e, 0, 0)),
        out_shape=jax.ShapeDtypeStruct((E, CAPACITY, D_MODEL), jnp.float32),
        scratch_shapes=[pltpu.VMEM((CAPACITY, D_MODEL), jnp.float32)],
    )(bufs_bf, w1b, b1, w2b, b2)


def _combine_body(slotc_hbm, gate_hbm, y_hbm, out_hbm,
                  slot_v, gate_v, rows_v, out_v, sem):
    cid = lax.axis_index("c")
    sid = lax.axis_index("s")
    wid = sid * 2 + cid
    base_e = wid * TPT * TOP_K

    pltpu.sync_copy(slotc_hbm.at[pl.ds(base_e, TPT * TOP_K)], slot_v)
    pltpu.sync_copy(gate_hbm.at[pl.ds(base_e, TPT * TOP_K)],
                    gate_v.at[pl.ds(0, TPT * TOP_K)])

    for k in range(4):
        pltpu.async_copy(
            y_hbm.at[slot_v.at[pl.ds(k * 32, 32)]], rows_v, sem).wait()

        def tok_i(i, _):
            gv = gate_v[pl.ds(k * 32 + 2 * i, 16)]
            g0 = gv[0]
            g1 = gv[1]

            def col_j(j, _):
                r0 = rows_v[2 * i, pl.ds(j * 16, 16)]
                r1 = rows_v[2 * i + 1, pl.ds(j * 16, 16)]
                out_v[i, pl.ds(j * 16, 16)] = g0 * r0 + g1 * r1
                return 0
            lax.fori_loop(0, D_MODEL // 16, col_j, 0)
            return 0
        lax.fori_loop(0, 16, tok_i, 0)
        pltpu.sync_copy(out_v, out_hbm.at[pl.ds(wid * TPT + k * 16, 16)])


def _combine(slotc_flat, gate_flat, y_flat):
    mesh = plsc.VectorSubcoreMesh(core_axis_name="c", subcore_axis_name="s", num_cores=2, num_subcores=16)
    return pl.kernel(
        _combine_body,
        out_type=jax.ShapeDtypeStruct((T, D_MODEL), jnp.float32),
        mesh=mesh,
        compiler_params=pltpu.CompilerParams(needs_layout_passes=False),
        scratch_types=[
            pltpu.VMEM((TPT * TOP_K,), jnp.int32),
            pltpu.VMEM((TPT * TOP_K + 32,), jnp.float32),
            pltpu.VMEM((32, D_MODEL), jnp.float32),
            pltpu.VMEM((16, D_MODEL), jnp.float32),
            pltpu.SemaphoreType.DMA,
        ],
    )(slotc_flat, gate_flat, y_flat)


def kernel(x, Wg, w1, b1, w2, b2):
    slotd, slotc, gate = _router(x, Wg)
    buffers = _dispatch(slotd.reshape(-1), x)
    bufs_bf = buffers.reshape(E, CAPACITY, D_MODEL).astype(jnp.bfloat16)
    y = _ffn(bufs_bf, w1.astype(jnp.bfloat16), b1[:, None, :],
             w2.astype(jnp.bfloat16), b2[:, None, :])
    out = _combine(slotc.reshape(-1), gate.reshape(-1),
                   y.reshape(NSLOT, D_MODEL))
    return out
